# baseline (speedup 1.0000x reference)
import jax
import jax.numpy as jnp
from jax.experimental import pallas as pl

D_IN = 1024
D_SAE = 32768
K = 64
TILE = 2048


def _enc_body(x_ref, bpre_ref, w_ref, benc_ref, z_ref):
    xm = x_ref[...] - bpre_ref[...]
    z = jnp.dot(xm, w_ref[...], preferred_element_type=jnp.float32)
    z_ref[...] = jnp.maximum(z + benc_ref[...], 0.0)


def kernel(x, b_pre, W_enc, b_enc, W_dec):
    G = D_SAE // TILE
    z = pl.pallas_call(
        _enc_body,
        grid=(G,),
        in_specs=[
            pl.BlockSpec((32, D_IN), lambda i: (0, 0)),
            pl.BlockSpec((1, D_IN), lambda i: (0, 0)),
            pl.BlockSpec((D_IN, TILE), lambda i: (0, i)),
            pl.BlockSpec((1, TILE), lambda i: (0, i)),
        ],
        out_specs=pl.BlockSpec((32, TILE), lambda i: (0, i)),
        out_shape=jax.ShapeDtypeStruct((32, D_SAE), jnp.float32),
    )(x, b_pre[None], W_enc, b_enc[None])
    vals, idx = jax.lax.top_k(z, K)
    z_sparse = jnp.zeros_like(z).at[jnp.arange(32)[:, None], idx].set(vals)
    x_hat = z_sparse @ W_dec + b_pre
    return (x_hat, z_sparse)


# speedup vs baseline: 2.1376x; 2.1376x over previous
import functools

import jax
import jax.numpy as jnp
from jax import lax
from jax.experimental import pallas as pl
from jax.experimental.pallas import tpu as pltpu
from jax.experimental.pallas import tpu_sc as plsc

D_IN = 1024
D_SAE = 32768
K = 64
TILE = 2048

NLANE = 16
NVREG = D_SAE // NLANE
CMAX = 2048


def _enc_body(x_ref, bpre_ref, w_ref, benc_ref, z_ref):
    xm = x_ref[...] - bpre_ref[...]
    z = jnp.dot(xm, w_ref[...], preferred_element_type=jnp.float32)
    z_ref[...] = jnp.maximum(z + benc_ref[...], 0.0)


def _encode(x, b_pre, W_enc, b_enc):
    G = D_SAE // TILE
    return pl.pallas_call(
        _enc_body,
        grid=(G,),
        in_specs=[
            pl.BlockSpec((32, D_IN), lambda i: (0, 0)),
            pl.BlockSpec((1, D_IN), lambda i: (0, 0)),
            pl.BlockSpec((D_IN, TILE), lambda i: (0, i)),
            pl.BlockSpec((1, TILE), lambda i: (0, i)),
        ],
        out_specs=pl.BlockSpec((32, TILE), lambda i: (0, i)),
        out_shape=jax.ShapeDtypeStruct((32, D_SAE), jnp.float32),
    )(x, b_pre[None], W_enc, b_enc[None])


def _splat(v):
    return jnp.full((NLANE,), v, jnp.float32)


def _mid(lo, hi):
    return lo + 0.5 * (hi - lo)


def _sc_body(z_hbm, wdec_hbm, bpre_hbm, zsp_hbm, xhat_hbm,
             zrow, cval, fvalp, fidxp, fidx, wrows, bprev, xrow,
             sem_g, sem_z):
    c = lax.axis_index("c")
    s = lax.axis_index("s")
    r = s * 2 + c

    pltpu.sync_copy(z_hbm.at[r], zrow)
    pltpu.sync_copy(bpre_hbm, bprev)

    iota16 = lax.iota(jnp.int32, NLANE)
    zero16f = jnp.zeros((NLANE,), jnp.float32)
    zero16i = jnp.zeros((NLANE,), jnp.int32)

    def amax_body(j, mv):
        return jnp.maximum(mv, zrow[pl.ds(j * NLANE, NLANE)])

    mv = lax.fori_loop(0, NVREG, amax_body, zero16f, unroll=8)
    M = jnp.max(mv)

    def count_full(t):
        tf = _splat(t)

        def b(j, cv):
            return cv + (zrow[pl.ds(j * NLANE, NLANE)] > tf).astype(jnp.int32)

        cv = lax.fori_loop(0, NVREG, b, zero16i, unroll=8)
        return jnp.sum(cv)

    tau0 = 0.5 * M
    c0 = count_full(tau0)

    def coarse_cond(st):
        lo, hi, tau, cnt = st
        in_range = jnp.logical_and(cnt >= K, cnt <= CMAX)
        mid = _mid(lo, hi)
        open_iv = jnp.logical_and(mid != lo, mid != hi)
        return jnp.logical_and(jnp.logical_not(in_range), open_iv)

    def coarse_body(st):
        lo, hi, tau, cnt = st
        lo = jnp.where(cnt > CMAX, tau, lo)
        hi = jnp.where(cnt < K, tau, hi)
        ntau = _mid(lo, hi)
        return (lo, hi, ntau, count_full(ntau))

    st = (jnp.float32(-1.0), M, tau0, c0)
    lo, hi, tau, cnt = lax.while_loop(coarse_cond, coarse_body, st)
    in_range = jnp.logical_and(cnt >= K, cnt <= CMAX)
    tauf = _splat(tau)

    def zc_body(j, _):
        cval[pl.ds(j * NLANE, NLANE)] = zero16f
        return 0

    lax.fori_loop(0, CMAX // NLANE, zc_body, 0, unroll=8)

    def compact_body(j, ptr):
        ok = jnp.full((NLANE,), ptr <= CMAX)
        v = zrow[pl.ds(j * NLANE, NLANE)]
        mask = jnp.logical_and(v > tauf, ok)
        plsc.store_compressed(cval.at[pl.ds(ptr, NLANE)], v, mask=mask)
        return ptr + jnp.sum(mask.astype(jnp.int32))

    lax.fori_loop(0, NVREG, compact_body, jnp.int32(0), unroll=4)

    lo2 = jnp.where(in_range, tau, hi)
    hi2 = jnp.where(in_range, M, hi)

    def count_cand(t):
        tf = _splat(t)

        def b(j, cv):
            return cv + (cval[pl.ds(j * NLANE, NLANE)] > tf).astype(jnp.int32)

        cv = lax.fori_loop(0, CMAX // NLANE, b, zero16i, unroll=8)
        return jnp.sum(cv)

    def fine_cond(st):
        lo, hi = st
        mid = _mid(lo, hi)
        return jnp.logical_and(mid != lo, mid != hi)

    def fine_body(st):
        lo, hi = st
        mid = _mid(lo, hi)
        cm = count_cand(mid)
        lo = jnp.where(cm >= K, mid, lo)
        hi = jnp.where(cm >= K, hi, mid)
        return (lo, hi)

    _, t = lax.while_loop(fine_cond, fine_body, (lo2, hi2))
    tf = _splat(t)
    m_eq = K - count_full(t)

    def extract_body(j, st):
        ptr, eq_seen = st
        v = zrow[pl.ds(j * NLANE, NLANE)]
        gt = v > tf
        eq = v == tf
        eqc = plsc.cumsum(eq.astype(jnp.int32))
        keep_eq = jnp.logical_and(eq, (eqc + eq_seen) <= m_eq)
        mask = jnp.logical_or(gt, keep_eq)
        idxv = iota16 + j * NLANE
        plsc.store_compressed(fvalp.at[pl.ds(ptr, NLANE)], v, mask=mask)
        plsc.store_compressed(fidxp.at[pl.ds(ptr, NLANE)], idxv, mask=mask)
        ptr = ptr + jnp.sum(mask.astype(jnp.int32))
        eq_seen = eq_seen + jnp.sum(eq.astype(jnp.int32))
        return (ptr, eq_seen)

    lax.fori_loop(0, NVREG, extract_body, (jnp.int32(0), jnp.int32(0)),
                  unroll=4)

    for g in range(K // NLANE):
        fidx[pl.ds(g * NLANE, NLANE)] = fidxp[pl.ds(g * NLANE, NLANE)]
    gather = pltpu.async_copy(wdec_hbm.at[fidx], wrows, sem_g)

    def zz_body(j, _):
        zrow[pl.ds(j * NLANE, NLANE)] = zero16f
        return 0

    lax.fori_loop(0, NVREG, zz_body, 0, unroll=8)
    for g in range(K // NLANE):
        idxv = fidx[pl.ds(g * NLANE, NLANE)]
        valv = fvalp[pl.ds(g * NLANE, NLANE)]
        plsc.store_scatter(zrow, [idxv], valv)
    zout = pltpu.async_copy(zrow, zsp_hbm.at[r], sem_z)

    gather.wait()

    NCH = 8
    for ch in range(NCH):
        base = ch * (D_IN // NCH)

        def jbody(j, accs):
            sv = fvalp[pl.ds(j, NLANE)][0]
            return tuple(
                a + sv * wrows[j, pl.ds(base + u * NLANE, NLANE)]
                for u, a in enumerate(accs)
            )

        accs = lax.fori_loop(
            0, K, jbody,
            tuple(zero16f for _ in range(D_IN // NCH // NLANE)), unroll=2)
        for u, a in enumerate(accs):
            off = base + u * NLANE
            xrow[pl.ds(off, NLANE)] = a + bprev[pl.ds(off, NLANE)]

    pltpu.sync_copy(xrow, xhat_hbm.at[r])
    zout.wait()


def _topk_decode(z, W_dec, b_pre):
    mesh = plsc.VectorSubcoreMesh(core_axis_name="c", subcore_axis_name="s")
    f32 = jnp.float32
    kfn = functools.partial(
        pl.kernel,
        out_type=(
            jax.ShapeDtypeStruct((32, D_SAE), f32),
            jax.ShapeDtypeStruct((32, D_IN), f32),
        ),
        mesh=mesh,
        compiler_params=pltpu.CompilerParams(needs_layout_passes=False),
        scratch_types=[
            pltpu.VMEM((D_SAE,), f32),
            pltpu.VMEM((CMAX + NLANE,), f32),
            pltpu.VMEM((K + NLANE,), f32),
            pltpu.VMEM((K + NLANE,), jnp.int32),
            pltpu.VMEM((K,), jnp.int32),
            pltpu.VMEM((K, D_IN), f32),
            pltpu.VMEM((D_IN,), f32),
            pltpu.VMEM((D_IN,), f32),
            pltpu.SemaphoreType.DMA,
            pltpu.SemaphoreType.DMA,
        ],
    )(_sc_body)
    return kfn(z, W_dec, b_pre)


def kernel(x, b_pre, W_enc, b_enc, W_dec):
    z = _encode(x, b_pre, W_enc, b_enc)
    z_sparse, x_hat = _topk_decode(z, W_dec, b_pre)
    return (x_hat, z_sparse)


# speedup vs baseline: 2.4779x; 1.1592x over previous
import functools

import jax
import jax.numpy as jnp
from jax import lax
from jax.experimental import pallas as pl
from jax.experimental.pallas import tpu as pltpu
from jax.experimental.pallas import tpu_sc as plsc

D_IN = 1024
D_SAE = 32768
K = 64
TILE = 2048

NLANE = 16
NVREG = D_SAE // NLANE
CMAX = 2048


def _enc_body(x_ref, bpre_ref, w_ref, benc_ref, z_ref):
    xm = x_ref[...] - bpre_ref[...]
    z = jnp.dot(xm, w_ref[...], preferred_element_type=jnp.float32)
    z_ref[...] = jnp.maximum(z + benc_ref[...], 0.0)


def _encode(x, b_pre, W_enc, b_enc):
    G = D_SAE // TILE
    return pl.pallas_call(
        _enc_body,
        grid=(G,),
        in_specs=[
            pl.BlockSpec((32, D_IN), lambda i: (0, 0)),
            pl.BlockSpec((1, D_IN), lambda i: (0, 0)),
            pl.BlockSpec((D_IN, TILE), lambda i: (0, i)),
            pl.BlockSpec((1, TILE), lambda i: (0, i)),
        ],
        out_specs=pl.BlockSpec((32, TILE), lambda i: (0, i)),
        out_shape=jax.ShapeDtypeStruct((32, D_SAE), jnp.float32),
    )(x, b_pre[None], W_enc, b_enc[None])


def _splat(v):
    return jnp.full((NLANE,), v, jnp.float32)


def _mid(lo, hi):
    return lo + 0.5 * (hi - lo)


def _sc_body(z_hbm, wdec_hbm, bpre_hbm, zsp_hbm, xhat_hbm,
             zrow, cval, fvalp, fidxp, fidx, wrows, bprev, xrow,
             sem_g, sem_z):
    c = lax.axis_index("c")
    s = lax.axis_index("s")
    r = s * 2 + c

    pltpu.sync_copy(z_hbm.at[r], zrow)
    pltpu.sync_copy(bpre_hbm, bprev)

    iota16 = lax.iota(jnp.int32, NLANE)
    zero16f = jnp.zeros((NLANE,), jnp.float32)
    zero16i = jnp.zeros((NLANE,), jnp.int32)

    def amax_body(j, mv):
        return jnp.maximum(mv, zrow[pl.ds(j * NLANE, NLANE)])

    mv = lax.fori_loop(0, NVREG, amax_body, zero16f, unroll=8)
    M = jnp.max(mv)

    def count_full(t):
        tf = _splat(t)

        def b(j, cv):
            return cv + (zrow[pl.ds(j * NLANE, NLANE)] > tf).astype(jnp.int32)

        cv = lax.fori_loop(0, NVREG, b, zero16i, unroll=8)
        return jnp.sum(cv)

    tau0 = 0.5 * M
    c0 = count_full(tau0)

    def coarse_cond(st):
        lo, hi, tau, cnt = st
        in_range = jnp.logical_and(cnt >= K, cnt <= CMAX)
        mid = _mid(lo, hi)
        open_iv = jnp.logical_and(mid != lo, mid != hi)
        return jnp.logical_and(jnp.logical_not(in_range), open_iv)

    def coarse_body(st):
        lo, hi, tau, cnt = st
        lo = jnp.where(cnt > CMAX, tau, lo)
        hi = jnp.where(cnt < K, tau, hi)
        ntau = _mid(lo, hi)
        return (lo, hi, ntau, count_full(ntau))

    st = (jnp.float32(-1.0), M, tau0, c0)
    lo, hi, tau, cnt = lax.while_loop(coarse_cond, coarse_body, st)
    in_range = jnp.logical_and(cnt >= K, cnt <= CMAX)
    tauf = _splat(tau)

    def zc_body(j, _):
        cval[pl.ds(j * NLANE, NLANE)] = zero16f
        return 0

    lax.fori_loop(0, CMAX // NLANE, zc_body, 0, unroll=8)

    @pl.when(in_range)
    def _():
        def compact_body(j, ptr):
            v = zrow[pl.ds(j * NLANE, NLANE)]
            mask = v > tauf
            plsc.store_compressed(cval.at[pl.ds(ptr, NLANE)], v, mask=mask)
            return ptr + plsc.all_reduce_population_count(mask)[0]

        lax.fori_loop(0, NVREG, compact_body, jnp.int32(0), unroll=8)

    lo2 = jnp.where(in_range, tau, hi)
    hi2 = jnp.where(in_range, M, hi)

    def count_cand(t):
        tf = _splat(t)

        def b(j, cv):
            return cv + (cval[pl.ds(j * NLANE, NLANE)] > tf).astype(jnp.int32)

        cv = lax.fori_loop(0, CMAX // NLANE, b, zero16i, unroll=8)
        return jnp.sum(cv)

    def fine_cond(st):
        lo, hi = st
        mid = _mid(lo, hi)
        return jnp.logical_and(mid != lo, mid != hi)

    def fine_body(st):
        lo, hi = st
        mid = _mid(lo, hi)
        cm = count_cand(mid)
        lo = jnp.where(cm >= K, mid, lo)
        hi = jnp.where(cm >= K, hi, mid)
        return (lo, hi)

    _, t = lax.while_loop(fine_cond, fine_body, (lo2, hi2))
    tf = _splat(t)
    cnt_gt = lax.cond(in_range, lambda: count_cand(t), lambda: count_full(t))
    m_eq = K - cnt_gt

    def extract_body(j, st):
        ptr, eq_seen = st
        v = zrow[pl.ds(j * NLANE, NLANE)]
        gt = v > tf
        eq = v == tf
        eqc = plsc.cumsum(eq.astype(jnp.int32))
        keep_eq = jnp.logical_and(eq, (eqc + eq_seen) <= m_eq)
        mask = jnp.logical_or(gt, keep_eq)
        idxv = iota16 + j * NLANE
        plsc.store_compressed(fvalp.at[pl.ds(ptr, NLANE)], v, mask=mask)
        plsc.store_compressed(fidxp.at[pl.ds(ptr, NLANE)], idxv, mask=mask)
        zrow[pl.ds(j * NLANE, NLANE)] = zero16f
        ptr = ptr + plsc.all_reduce_population_count(mask)[0]
        eq_seen = eq_seen + plsc.all_reduce_population_count(eq)[0]
        return (ptr, eq_seen)

    lax.fori_loop(0, NVREG, extract_body, (jnp.int32(0), jnp.int32(0)),
                  unroll=8)

    for g in range(K // NLANE):
        fidx[pl.ds(g * NLANE, NLANE)] = fidxp[pl.ds(g * NLANE, NLANE)]
    gather = pltpu.async_copy(wdec_hbm.at[fidx], wrows, sem_g)

    for g in range(K // NLANE):
        idxv = fidx[pl.ds(g * NLANE, NLANE)]
        valv = fvalp[pl.ds(g * NLANE, NLANE)]
        plsc.store_scatter(zrow, [idxv], valv)
    zout = pltpu.async_copy(zrow, zsp_hbm.at[r], sem_z)

    gather.wait()

    NCH = 8
    for ch in range(NCH):
        base = ch * (D_IN // NCH)

        def jbody(j, accs):
            sv = fvalp[pl.ds(j, NLANE)][0]
            return tuple(
                a + sv * wrows[j, pl.ds(base + u * NLANE, NLANE)]
                for u, a in enumerate(accs)
            )

        accs = lax.fori_loop(
            0, K, jbody,
            tuple(zero16f for _ in range(D_IN // NCH // NLANE)), unroll=2)
        for u, a in enumerate(accs):
            off = base + u * NLANE
            xrow[pl.ds(off, NLANE)] = a + bprev[pl.ds(off, NLANE)]

    pltpu.sync_copy(xrow, xhat_hbm.at[r])
    zout.wait()


def _topk_decode(z, W_dec, b_pre):
    mesh = plsc.VectorSubcoreMesh(core_axis_name="c", subcore_axis_name="s")
    f32 = jnp.float32
    kfn = functools.partial(
        pl.kernel,
        out_type=(
            jax.ShapeDtypeStruct((32, D_SAE), f32),
            jax.ShapeDtypeStruct((32, D_IN), f32),
        ),
        mesh=mesh,
        compiler_params=pltpu.CompilerParams(needs_layout_passes=False),
        scratch_types=[
            pltpu.VMEM((D_SAE,), f32),
            pltpu.VMEM((CMAX + NLANE,), f32),
            pltpu.VMEM((K + NLANE,), f32),
            pltpu.VMEM((K + NLANE,), jnp.int32),
            pltpu.VMEM((K,), jnp.int32),
            pltpu.VMEM((K, D_IN), f32),
            pltpu.VMEM((D_IN,), f32),
            pltpu.VMEM((D_IN,), f32),
            pltpu.SemaphoreType.DMA,
            pltpu.SemaphoreType.DMA,
        ],
    )(_sc_body)
    return kfn(z, W_dec, b_pre)


def kernel(x, b_pre, W_enc, b_enc, W_dec):
    z = _encode(x, b_pre, W_enc, b_enc)
    z_sparse, x_hat = _topk_decode(z, W_dec, b_pre)
    return (x_hat, z_sparse)


# speedup vs baseline: 2.8131x; 1.1353x over previous
import functools

import jax
import jax.numpy as jnp
from jax import lax
from jax.experimental import pallas as pl
from jax.experimental.pallas import tpu as pltpu
from jax.experimental.pallas import tpu_sc as plsc

D_IN = 1024
D_SAE = 32768
K = 64
TILE = 2048

NLANE = 16
NVREG = D_SAE // NLANE
CMAX = 2048
CAP = 256
ECAP = 192


def _enc_body(x_ref, bpre_ref, w_ref, benc_ref, z_ref):
    xm = x_ref[...] - bpre_ref[...]
    z = jnp.dot(xm, w_ref[...], preferred_element_type=jnp.float32)
    z_ref[...] = jnp.maximum(z + benc_ref[...], 0.0)


def _encode(x, b_pre, W_enc, b_enc):
    G = D_SAE // TILE
    return pl.pallas_call(
        _enc_body,
        grid=(G,),
        in_specs=[
            pl.BlockSpec((32, D_IN), lambda i: (0, 0)),
            pl.BlockSpec((1, D_IN), lambda i: (0, 0)),
            pl.BlockSpec((D_IN, TILE), lambda i: (0, i)),
            pl.BlockSpec((1, TILE), lambda i: (0, i)),
        ],
        out_specs=pl.BlockSpec((32, TILE), lambda i: (0, i)),
        out_shape=jax.ShapeDtypeStruct((32, D_SAE), jnp.float32),
    )(x, b_pre[None], W_enc, b_enc[None])


def _splat(v):
    return jnp.full((NLANE,), v, jnp.float32)


def _mid(lo, hi):
    return lo + 0.5 * (hi - lo)


def _sc_body(z_hbm, wdec_hbm, bpre_hbm, zsp_hbm, xhat_hbm,
             zrow, cslot, eidx, evalb, fvalp, fidxp, fidx, wrows, bprev,
             xrow, sem_g, sem_z):
    c = lax.axis_index("c")
    s = lax.axis_index("s")
    r = s * 2 + c

    pltpu.sync_copy(z_hbm.at[r], zrow)
    pltpu.sync_copy(bpre_hbm, bprev)

    iota16 = lax.iota(jnp.int32, NLANE)
    zero16f = jnp.zeros((NLANE,), jnp.float32)
    zero16i = jnp.zeros((NLANE,), jnp.int32)

    def amax_body(j, mv):
        return jnp.maximum(mv, zrow[pl.ds(j * NLANE, NLANE)])

    mv = lax.fori_loop(0, NVREG, amax_body, zero16f, unroll=8)
    M = jnp.max(mv)

    def count_full(t):
        tf = _splat(t)

        def b(j, cv):
            return cv + (zrow[pl.ds(j * NLANE, NLANE)] > tf).astype(jnp.int32)

        cv = lax.fori_loop(0, NVREG, b, zero16i, unroll=8)
        return jnp.sum(cv)

    tau0 = 0.5 * M
    c0 = count_full(tau0)

    def coarse_cond(st):
        lo, hi, tau, cnt = st
        in_range = jnp.logical_and(cnt >= K, cnt <= CMAX)
        mid = _mid(lo, hi)
        open_iv = jnp.logical_and(mid != lo, mid != hi)
        return jnp.logical_and(jnp.logical_not(in_range), open_iv)

    def coarse_body(st):
        lo, hi, tau, cnt = st
        lo = jnp.where(cnt > CMAX, tau, lo)
        hi = jnp.where(cnt < K, tau, hi)
        ntau = _mid(lo, hi)
        return (lo, hi, ntau, count_full(ntau))

    st = (jnp.float32(-1.0), M, tau0, c0)
    lo, hi, tau, cnt = lax.while_loop(coarse_cond, coarse_body, st)
    in_range = jnp.logical_and(cnt >= K, cnt <= CMAX)
    tauf = _splat(tau)

    def compact_body(j, cnt_v):
        v = zrow[pl.ds(j * NLANE, NLANE)]
        mask = v > tauf
        dst = jnp.minimum(cnt_v, CAP) * NLANE + iota16
        plsc.store_scatter(cslot, [dst], v, mask=mask)
        return cnt_v + mask.astype(jnp.int32)

    cnt_v = lax.fori_loop(0, NVREG, compact_body, zero16i, unroll=8)
    maxc = jnp.max(cnt_v)
    use_cand = jnp.logical_and(in_range, maxc <= CAP)
    nblk = lax.div(jnp.minimum(maxc, CAP) + 3, 4)

    def count_cand(t):
        tf2 = _splat(t)

        def b(i, cv):
            for k in range(4):
                sl = i * 4 + k
                v = cslot[pl.ds(sl * NLANE, NLANE)]
                ok = jnp.logical_and(v > tf2, cnt_v > sl)
                cv = cv + ok.astype(jnp.int32)
            return cv

        cv = lax.fori_loop(0, nblk, b, zero16i)
        return jnp.sum(cv)

    def count_sel(t):
        return lax.cond(use_cand, lambda: count_cand(t),
                        lambda: count_full(t))

    lo2 = jnp.where(in_range, tau, hi)
    hi2 = jnp.where(in_range, M, hi)

    def fine_cond(st):
        lo_, hi_ = st
        mid = _mid(lo_, hi_)
        return jnp.logical_and(mid != lo_, mid != hi_)

    def fine_body(st):
        lo_, hi_ = st
        mid = _mid(lo_, hi_)
        cm = count_sel(mid)
        lo_ = jnp.where(cm >= K, mid, lo_)
        hi_ = jnp.where(cm >= K, hi_, mid)
        return (lo_, hi_)

    _, t = lax.while_loop(fine_cond, fine_body, (lo2, hi2))
    tf = _splat(t)
    cnt_gt = count_sel(t)
    m_eq = K - cnt_gt

    def ext_body(j, ecnt_v):
        v = zrow[pl.ds(j * NLANE, NLANE)]
        gt = v > tf
        eq = jnp.logical_and(v == tf, ecnt_v < 128)
        mask = jnp.logical_or(gt, eq)
        dst = jnp.minimum(ecnt_v, ECAP) * NLANE + iota16
        idxv = iota16 + j * NLANE
        plsc.store_scatter(eidx, [dst], idxv, mask=mask)
        plsc.store_scatter(evalb, [dst], v, mask=mask)
        zrow[pl.ds(j * NLANE, NLANE)] = zero16f
        return ecnt_v + mask.astype(jnp.int32)

    ecnt_v = lax.fori_loop(0, NVREG, ext_body, zero16i, unroll=8)
    emax = jnp.max(ecnt_v)
    neblk = lax.div(jnp.minimum(emax, ECAP) + 3, 4)

    def count_eq_le(ithr):
        it = jnp.full((NLANE,), ithr, jnp.int32)

        def b(i, cv):
            for k in range(4):
                sl = i * 4 + k
                ev = evalb[pl.ds(sl * NLANE, NLANE)]
                ei = eidx[pl.ds(sl * NLANE, NLANE)]
                ok = jnp.logical_and(ev == tf, ecnt_v > sl)
                ok = jnp.logical_and(ok, ei <= it)
                cv = cv + ok.astype(jnp.int32)
            return cv

        cv = lax.fori_loop(0, neblk, b, zero16i)
        return jnp.sum(cv)

    def eq_cond(st):
        lo_, hi_ = st
        return hi_ - lo_ > 1

    def eq_body(st):
        lo_, hi_ = st
        mid = lax.div(lo_ + hi_, 2)
        ce = count_eq_le(mid)
        lo_ = jnp.where(ce < m_eq, mid, lo_)
        hi_ = jnp.where(ce < m_eq, hi_, mid)
        return (lo_, hi_)

    _, ithr = lax.while_loop(eq_cond, eq_body,
                             (jnp.int32(-1), jnp.int32(D_SAE)))
    itf = jnp.full((NLANE,), ithr, jnp.int32)

    def fc_body(i, ptr):
        for k in range(4):
            sl = i * 4 + k
            ev = evalb[pl.ds(sl * NLANE, NLANE)]
            ei = eidx[pl.ds(sl * NLANE, NLANE)]
            valid = ecnt_v > sl
            gtm = jnp.logical_and(ev > tf, valid)
            eqm = jnp.logical_and(jnp.logical_and(ev == tf, valid),
                                  ei <= itf)
            m = jnp.logical_or(gtm, eqm)
            plsc.store_compressed(fvalp.at[pl.ds(ptr, NLANE)], ev, mask=m)
            plsc.store_compressed(fidxp.at[pl.ds(ptr, NLANE)], ei, mask=m)
            ptr = ptr + plsc.all_reduce_population_count(m)[0]
        return ptr

    lax.fori_loop(0, neblk, fc_body, jnp.int32(0))

    for g in range(K // NLANE):
        fidx[pl.ds(g * NLANE, NLANE)] = fidxp[pl.ds(g * NLANE, NLANE)]
    gather = pltpu.async_copy(wdec_hbm.at[fidx], wrows, sem_g)

    for g in range(K // NLANE):
        idxv = fidx[pl.ds(g * NLANE, NLANE)]
        valv = fvalp[pl.ds(g * NLANE, NLANE)]
        plsc.store_scatter(zrow, [idxv], valv)
    zout = pltpu.async_copy(zrow, zsp_hbm.at[r], sem_z)

    gather.wait()

    NCH = 8
    for ch in range(NCH):
        base = ch * (D_IN // NCH)

        def jbody(j, accs):
            sv = fvalp[pl.ds(j, NLANE)][0]
            return tuple(
                a + sv * wrows[j, pl.ds(base + u * NLANE, NLANE)]
                for u, a in enumerate(accs)
            )

        accs = lax.fori_loop(
            0, K, jbody,
            tuple(zero16f for _ in range(D_IN // NCH // NLANE)), unroll=2)
        for u, a in enumerate(accs):
            off = base + u * NLANE
            xrow[pl.ds(off, NLANE)] = a + bprev[pl.ds(off, NLANE)]

    pltpu.sync_copy(xrow, xhat_hbm.at[r])
    zout.wait()


def _topk_decode(z, W_dec, b_pre):
    mesh = plsc.VectorSubcoreMesh(core_axis_name="c", subcore_axis_name="s")
    f32 = jnp.float32
    kfn = functools.partial(
        pl.kernel,
        out_type=(
            jax.ShapeDtypeStruct((32, D_SAE), f32),
            jax.ShapeDtypeStruct((32, D_IN), f32),
        ),
        mesh=mesh,
        compiler_params=pltpu.CompilerParams(needs_layout_passes=False),
        scratch_types=[
            pltpu.VMEM((D_SAE,), f32),
            pltpu.VMEM(((CAP + 4) * NLANE,), f32),
            pltpu.VMEM(((ECAP + 4) * NLANE,), jnp.int32),
            pltpu.VMEM(((ECAP + 4) * NLANE,), f32),
            pltpu.VMEM((K + NLANE,), f32),
            pltpu.VMEM((K + NLANE,), jnp.int32),
            pltpu.VMEM((K,), jnp.int32),
            pltpu.VMEM((K, D_IN), f32),
            pltpu.VMEM((D_IN,), f32),
            pltpu.VMEM((D_IN,), f32),
            pltpu.SemaphoreType.DMA,
            pltpu.SemaphoreType.DMA,
        ],
    )(_sc_body)
    return kfn(z, W_dec, b_pre)


def kernel(x, b_pre, W_enc, b_enc, W_dec):
    z = _encode(x, b_pre, W_enc, b_enc)
    z_sparse, x_hat = _topk_decode(z, W_dec, b_pre)
    return (x_hat, z_sparse)
